# Initial kernel scaffold; baseline (speedup 1.0000x reference)
import jax
import jax.numpy as jnp
from jax.experimental import pallas as pl


def kernel(user_id, item_id, category_id, region_id, offset, E_category, E_item, E_region, E_user, W1, b1, W_out, b_out):
    raise NotImplementedError("write your pallas kernel here")


# speedup vs baseline: 3.6539x; 3.6539x over previous
import functools

import jax
import jax.numpy as jnp
from jax import lax
from jax.experimental import pallas as pl
from jax.experimental.pallas import tpu as pltpu
from jax.experimental.pallas import tpu_sc as plsc

B = 16384
D = 50
HIDDEN = 64
NC = 2
NS = 16
NW = NC * NS
BPW = B // NW
NCHUNK = 4
CHUNK = BPW // NCHUNK


def _sc_gather4(t0, t1, t2, t3, i0, i1, i2, i3):
    mesh = plsc.VectorSubcoreMesh(core_axis_name="c", subcore_axis_name="s")
    out_type = tuple(jax.ShapeDtypeStruct((B, D), jnp.float32) for _ in range(4))
    scratch = [
        pltpu.VMEM((NCHUNK, CHUNK), jnp.int32),
        pltpu.VMEM((BPW, D), jnp.float32),
        pltpu.VMEM((NCHUNK, CHUNK), jnp.int32),
        pltpu.VMEM((BPW, D), jnp.float32),
        pltpu.VMEM((NCHUNK, CHUNK), jnp.int32),
        pltpu.VMEM((BPW, D), jnp.float32),
        pltpu.VMEM((NCHUNK, CHUNK), jnp.int32),
        pltpu.VMEM((BPW, D), jnp.float32),
        pltpu.SemaphoreType.DMA,
    ]

    @functools.partial(pl.kernel, mesh=mesh, out_type=out_type,
                       scratch_types=scratch,
                       compiler_params=pltpu.CompilerParams(
                           use_tc_tiling_on_sc=False))
    def k(t0, t1, t2, t3, i0, i1, i2, i3,
          o0, o1, o2, o3,
          vi0, vr0, vi1, vr1, vi2, vr2, vi3, vr3, sem):
        wid = lax.axis_index("s") * NC + lax.axis_index("c")
        base = wid * BPW
        tabs = (t0, t1, t2, t3)
        idxs = (i0, i1, i2, i3)
        outs = (o0, o1, o2, o3)
        vis = (vi0, vi1, vi2, vi3)
        vrs = (vr0, vr1, vr2, vr3)
        for t in range(4):
            pltpu.sync_copy(idxs[t].at[wid], vis[t])
        handles = []
        for t in range(4):
            for c in range(NCHUNK):
                handles.append(pltpu.async_copy(
                    tabs[t].at[vis[t].at[c]],
                    vrs[t].at[pl.ds(c * CHUNK, CHUNK), :],
                    sem))
        for h in handles:
            h.wait()
        for t in range(4):
            pltpu.sync_copy(vrs[t], outs[t].at[pl.ds(base, BPW)])

    return k(t0, t1, t2, t3, i0, i1, i2, i3)


def _tc_mlp(xc, xi, xr, xu, w1c, w1i, w1r, w1u, b1, wo, bo, offset):
    bm = 2048
    grid = (B // bm,)

    def body(xc_ref, xi_ref, xr_ref, xu_ref, w1c_ref, w1i_ref, w1r_ref,
             w1u_ref, b1_ref, wo_ref, bo_ref, off_ref, out_ref):
        h = jnp.dot(xc_ref[...], w1c_ref[...], preferred_element_type=jnp.float32)
        h = h + jnp.dot(xi_ref[...], w1i_ref[...], preferred_element_type=jnp.float32)
        h = h + jnp.dot(xr_ref[...], w1r_ref[...], preferred_element_type=jnp.float32)
        h = h + jnp.dot(xu_ref[...], w1u_ref[...], preferred_element_type=jnp.float32)
        h = jnp.maximum(h + b1_ref[...], 0.0)
        y = jnp.sum(h * wo_ref[...], axis=1) + bo_ref[0, 0]
        out_ref[...] = y + off_ref[...]

    x_spec = pl.BlockSpec((bm, D), lambda i: (i, 0))
    w_spec = pl.BlockSpec((D, HIDDEN), lambda i: (0, 0))
    return pl.pallas_call(
        body,
        grid=grid,
        in_specs=[
            x_spec, x_spec, x_spec, x_spec,
            w_spec, w_spec, w_spec, w_spec,
            pl.BlockSpec((1, HIDDEN), lambda i: (0, 0)),
            pl.BlockSpec((1, HIDDEN), lambda i: (0, 0)),
            pl.BlockSpec((1, 1), lambda i: (0, 0)),
            pl.BlockSpec((bm,), lambda i: (i,)),
        ],
        out_specs=pl.BlockSpec((bm,), lambda i: (i,)),
        out_shape=jax.ShapeDtypeStruct((B,), jnp.float32),
    )(xc, xi, xr, xu, w1c, w1i, w1r, w1u, b1, wo, bo, offset)


def kernel(user_id, item_id, category_id, region_id, offset,
           E_category, E_item, E_region, E_user, W1, b1, W_out, b_out):
    ic = category_id.astype(jnp.int32).reshape(NW, NCHUNK, CHUNK)
    ii = item_id.astype(jnp.int32).reshape(NW, NCHUNK, CHUNK)
    ir = region_id.astype(jnp.int32).reshape(NW, NCHUNK, CHUNK)
    iu = user_id.astype(jnp.int32).reshape(NW, NCHUNK, CHUNK)
    xc = jnp.take(E_category, category_id, axis=0)
    xi = jnp.take(E_item, item_id, axis=0)
    xr = jnp.take(E_region, region_id, axis=0)
    xu = jnp.take(E_user, user_id, axis=0)
    w1c = W1[0:D]
    w1i = W1[D:2 * D]
    w1r = W1[2 * D:3 * D]
    w1u = W1[3 * D:4 * D]
    return _tc_mlp(xc, xi, xr, xu, w1c, w1i, w1r, w1u,
                   b1.reshape(1, HIDDEN), W_out.reshape(1, HIDDEN),
                   b_out.reshape(1, 1), offset)
